# baseline (speedup 1.0000x reference)
import functools

import jax
import jax.numpy as jnp
from jax import lax
from jax.experimental import pallas as pl
from jax.experimental.pallas import tpu as pltpu
from jax.experimental.pallas import tpu_sc as plsc

N = 10000
E = 320000
G = 64
NC, NS = 2, 16
NW = NC * NS
EPT = 10240
E_PAD = EPT * NW
R_TILE = 640
N_ROWS = NS * R_TILE
TRASH = N_ROWS - 8
ZROWS = 64
F32 = jnp.float32


def _sc_agg(D, gather, K, NBUF):
  CPT = EPT // K
  mesh = plsc.VectorSubcoreMesh(core_axis_name="c", subcore_axis_name="s")
  scratch = [
      pltpu.VMEM((CPT, K), jnp.int32),
      pltpu.VMEM((K, D), F32),
      pltpu.VMEM((ZROWS, D), F32),
      pltpu.VMEM_SHARED((N_ROWS, D), F32),
      pltpu.SemaphoreType.DMA,
  ]
  if gather:
    scratch = ([pltpu.VMEM((CPT, K), jnp.int32)] + scratch
               + [pltpu.VMEM((K, D), F32)] * (NBUF - 1)
               + [pltpu.SemaphoreType.DMA] * (2 * NBUF - 1))

  @functools.partial(
      pl.kernel,
      out_type=jax.ShapeDtypeStruct((NC, N_ROWS, D), F32),
      mesh=mesh,
      scratch_types=scratch,
      compiler_params=pltpu.CompilerParams(use_tc_tiling_on_sc=False),
  )
  def body(*refs):
    if gather:
      (u_hbm, src_hbm, dst_hbm, out_hbm,
       idx_s, idx_d, buf, zbuf, acc, sem) = refs[:10]
      bufs = (buf,) + refs[10:10 + NBUF - 1]
      gsems = (sem,) + refs[10 + NBUF - 1:10 + 2 * (NBUF - 1)]
      ssems = refs[10 + 2 * (NBUF - 1):]
    else:
      dst_hbm, out_hbm, idx_d, buf, zbuf, acc, sem = refs
    c = lax.axis_index("c")
    s = lax.axis_index("s")
    wid = c * NS + s

    nvec = D // 16

    def fill(ref, rows, val):
      def w(i, _):
        r = i // nvec
        col = (i % nvec) * 16
        ref[r, pl.ds(col, 16)] = jnp.full((16,), val, F32)
        return 0
      lax.fori_loop(0, rows * nvec, w, 0)

    fill(zbuf, ZROWS, 0.0)
    def zcopy(k, _):
      pltpu.sync_copy(zbuf, acc.at[pl.ds(s * R_TILE + k * ZROWS, ZROWS)])
      return 0
    lax.fori_loop(0, R_TILE // ZROWS, zcopy, 0)
    if not gather:
      fill(buf, K, 1.0)
    pltpu.sync_copy(dst_hbm.at[pl.ds(wid * CPT, CPT)], idx_d)
    if gather:
      pltpu.sync_copy(src_hbm.at[pl.ds(wid * CPT, CPT)], idx_s)
    plsc.subcore_barrier()

    if gather:
      NG = CPT // NBUF

      def gath(b, t):
        return pltpu.make_async_copy(u_hbm.at[idx_s.at[b + t]],
                                     bufs[t], gsems[t])

      def scat(b, t):
        return pltpu.make_async_copy(bufs[t], acc.at[idx_d.at[b + t]],
                                     ssems[t])

      for t in range(NBUF):
        gath(0, t).start()

      def group(j, _):
        base = j * NBUF
        for t in range(NBUF):
          gath(base, t).wait()
          scat(base, t).start(add=True)

        @pl.when(j < NG - 1)
        def _():
          for t in range(NBUF):
            scat(base, t).wait()
            gath(base + NBUF, t).start()
        return 0
      lax.fori_loop(0, NG, group, 0)
      for t in range(NBUF):
        scat((NG - 1) * NBUF, t).wait()
    else:
      def chunk(i, _):
        pltpu.sync_copy(buf, acc.at[idx_d.at[i]], add=True)
        return 0
      lax.fori_loop(0, CPT, chunk, 0)
    plsc.subcore_barrier()
    pltpu.sync_copy(acc.at[pl.ds(s * R_TILE, R_TILE)],
                    out_hbm.at[c, pl.ds(s * R_TILE, R_TILE)])

  return body


_deg_sc = _sc_agg(16, gather=False, K=256, NBUF=1)
_agg64_sc = _sc_agg(64, gather=True, K=256, NBUF=2)
_agg16_sc = _sc_agg(16, gather=True, K=256, NBUF=8)


def _tc(body, out_shapes):
  return pl.pallas_call(body, out_shape=out_shapes)


def _mm_body(x_ref, w_ref, o_ref):
  o_ref[...] = jnp.dot(x_ref[...], w_ref[...], preferred_element_type=F32)


def _prep_body(dp_ref, xw_ref, dis_ref, u_ref):
  deg = (jnp.max(dp_ref[0, :N, :], axis=1, keepdims=True)
         + jnp.max(dp_ref[1, :N, :], axis=1, keepdims=True) + 1.0)
  dis = lax.rsqrt(jnp.maximum(deg, 1e-12))
  dis_ref[...] = dis
  u_ref[...] = xw_ref[...] * dis


def _bn_relu(t, g, be):
  mu = jnp.mean(t, axis=0, keepdims=True)
  xc = t - mu
  var = jnp.mean(xc * xc, axis=0, keepdims=True)
  return jnp.maximum(xc * lax.rsqrt(var + 1e-5) * g + be, 0.0)


def _l1_body(p_ref, u1_ref, dis_ref, b_ref, g_ref, be_ref, u2_ref):
  dis = dis_ref[...]
  t = dis * (p_ref[0, :N, :] + p_ref[1, :N, :] + u1_ref[...]) + b_ref[...]
  u2_ref[...] = _bn_relu(t, g_ref[...], be_ref[...]) * dis


def _l2_body(q_ref, u2_ref, dis_ref, w2_ref, b2_ref, g2_ref, be2_ref,
             w3_ref, u3_ref):
  dis = dis_ref[...]
  aggpre = dis * (q_ref[0, :N, :] + q_ref[1, :N, :] + u2_ref[...])
  t = jnp.dot(aggpre, w2_ref[...], preferred_element_type=F32) + b2_ref[...]
  h2 = _bn_relu(t, g2_ref[...], be2_ref[...])
  u3_ref[...] = jnp.dot(h2, w3_ref[...], preferred_element_type=F32) * dis


def _fin_body(r_ref, u3_ref, dis_ref, b3_ref, batch_ref, o_ref):
  h3 = dis_ref[...] * (r_ref[0, :N, :] + r_ref[1, :N, :] + u3_ref[...]) \
      + b3_ref[...]
  gid = lax.broadcasted_iota(jnp.int32, (N, G), 1)
  oh = (gid == batch_ref[...]).astype(F32)
  dn = (((0,), (0,)), ((), ()))
  sums = lax.dot_general(oh, h3, dn, preferred_element_type=F32)
  counts = lax.dot_general(oh, jnp.ones((N, 1), F32), dn,
                           preferred_element_type=F32)
  pooled = sums / jnp.maximum(counts, 1.0)
  lane = lax.broadcasted_iota(jnp.int32, (G, 16), 1)
  valid = lane < 10
  xm = jnp.where(valid, pooled, -1e30)
  mx = jnp.max(xm, axis=1, keepdims=True)
  ex = jnp.where(valid, jnp.exp(xm - mx), 0.0)
  lse = jnp.log(jnp.sum(ex, axis=1, keepdims=True))
  o_ref[...] = xm - mx - lse


def kernel(x, edge_index, batch, W1, b1, g1, be1, W2, b2, g2, be2, W3, b3):
  src = edge_index[0]
  dst = edge_index[1]
  pad = E_PAD - E
  srcp = jnp.concatenate([src, jnp.zeros((pad,), jnp.int32)]).reshape(-1, 256)
  dstp = jnp.concatenate([dst, jnp.full((pad,), TRASH, jnp.int32)]
                         ).reshape(-1, 256)

  dp = _deg_sc(dstp)
  xw1 = _tc(_mm_body, jax.ShapeDtypeStruct((N, 64), F32))(x, W1)
  dis, u1 = _tc(_prep_body, (jax.ShapeDtypeStruct((N, 1), F32),
                             jax.ShapeDtypeStruct((N, 64), F32)))(dp, xw1)
  p = _agg64_sc(u1, srcp, dstp)
  u2 = _tc(_l1_body, jax.ShapeDtypeStruct((N, 64), F32))(
      p, u1, dis, b1.reshape(1, -1), g1.reshape(1, -1), be1.reshape(1, -1))
  q = _agg64_sc(u2, srcp, dstp)
  w3p = jnp.pad(W3, ((0, 0), (0, 6)))
  u3 = _tc(_l2_body, jax.ShapeDtypeStruct((N, 16), F32))(
      q, u2, dis, W2, b2.reshape(1, -1), g2.reshape(1, -1),
      be2.reshape(1, -1), w3p)
  r = _agg16_sc(u3, srcp, dstp)
  b3p = jnp.pad(b3, (0, 6)).reshape(1, -1)
  out = _tc(_fin_body, jax.ShapeDtypeStruct((G, 16), F32))(
      r, u3, dis, b3p, batch[:, None])
  return out[:, :10]


# speedup vs baseline: 1.5495x; 1.5495x over previous
import functools

import jax
import jax.numpy as jnp
from jax import lax
from jax.experimental import pallas as pl
from jax.experimental.pallas import tpu as pltpu
from jax.experimental.pallas import tpu_sc as plsc

N = 10000
E = 320000
G = 64
NC, NS = 2, 16
NW = NC * NS
EPT = 10240
E_PAD = EPT * NW
R_TILE = 640
N_ROWS = NS * R_TILE
TRASH = N_ROWS - 8
ZROWS = 64
F32 = jnp.float32


def _sc_agg(D, gather, K, NBUF, dtype=F32):
  CPT = EPT // K
  lanes = 32 if dtype == jnp.bfloat16 else 16
  mesh = plsc.VectorSubcoreMesh(core_axis_name="c", subcore_axis_name="s")
  scratch = [
      pltpu.VMEM((CPT, K), jnp.int32),
      pltpu.VMEM((K, D), dtype),
      pltpu.VMEM((ZROWS, D), dtype),
      pltpu.VMEM_SHARED((N_ROWS, D), dtype),
      pltpu.SemaphoreType.DMA,
  ]
  if gather:
    scratch = ([pltpu.VMEM((CPT, K), jnp.int32)] + scratch
               + [pltpu.VMEM((K, D), dtype)] * (NBUF - 1)
               + [pltpu.SemaphoreType.DMA] * (2 * NBUF - 1))

  @functools.partial(
      pl.kernel,
      out_type=jax.ShapeDtypeStruct((NC, N_ROWS, D), dtype),
      mesh=mesh,
      scratch_types=scratch,
      compiler_params=pltpu.CompilerParams(use_tc_tiling_on_sc=False),
  )
  def body(*refs):
    if gather:
      (u_hbm, src_hbm, dst_hbm, out_hbm,
       idx_s, idx_d, buf, zbuf, acc, sem) = refs[:10]
      bufs = (buf,) + refs[10:10 + NBUF - 1]
      gsems = (sem,) + refs[10 + NBUF - 1:10 + 2 * (NBUF - 1)]
      ssems = refs[10 + 2 * (NBUF - 1):]
    else:
      dst_hbm, out_hbm, idx_d, buf, zbuf, acc, sem = refs
    c = lax.axis_index("c")
    s = lax.axis_index("s")
    wid = c * NS + s

    nvec = D // lanes

    def fill(ref, rows, val):
      def w(i, _):
        r = i // nvec
        col = (i % nvec) * lanes
        ref[r, pl.ds(col, lanes)] = jnp.full((lanes,), val, dtype)
        return 0
      lax.fori_loop(0, rows * nvec, w, 0)

    fill(zbuf, ZROWS, 0.0)
    def zcopy(k, _):
      pltpu.sync_copy(zbuf, acc.at[pl.ds(s * R_TILE + k * ZROWS, ZROWS)])
      return 0
    lax.fori_loop(0, R_TILE // ZROWS, zcopy, 0)
    if not gather:
      fill(buf, K, 1.0)
    pltpu.sync_copy(dst_hbm.at[pl.ds(wid * CPT, CPT)], idx_d)
    if gather:
      pltpu.sync_copy(src_hbm.at[pl.ds(wid * CPT, CPT)], idx_s)
    plsc.subcore_barrier()

    if gather:
      NG = CPT // NBUF

      def gath(b, t):
        return pltpu.make_async_copy(u_hbm.at[idx_s.at[b + t]],
                                     bufs[t], gsems[t])

      def scat(b, t):
        return pltpu.make_async_copy(bufs[t], acc.at[idx_d.at[b + t]],
                                     ssems[t])

      for t in range(NBUF):
        gath(0, t).start()

      def group(j, _):
        base = j * NBUF
        for t in range(NBUF):
          gath(base, t).wait()
          scat(base, t).start(add=True)

        @pl.when(j < NG - 1)
        def _():
          for t in range(NBUF):
            scat(base, t).wait()
            gath(base + NBUF, t).start()
        return 0
      lax.fori_loop(0, NG, group, 0)
      for t in range(NBUF):
        scat((NG - 1) * NBUF, t).wait()
    else:
      def chunk(i, _):
        pltpu.sync_copy(buf, acc.at[idx_d.at[i]], add=True)
        return 0
      lax.fori_loop(0, CPT, chunk, 0)
    plsc.subcore_barrier()
    pltpu.sync_copy(acc.at[pl.ds(s * R_TILE, R_TILE)],
                    out_hbm.at[c, pl.ds(s * R_TILE, R_TILE)])

  return body


_deg_sc = _sc_agg(16, gather=False, K=256, NBUF=1)
_agg64_sc = _sc_agg(64, gather=True, K=256, NBUF=4, dtype=jnp.bfloat16)
_agg16_sc = _sc_agg(16, gather=True, K=256, NBUF=8)


def _tc(body, out_shapes):
  return pl.pallas_call(body, out_shape=out_shapes)


def _mm_body(x_ref, w_ref, o_ref):
  o_ref[...] = jnp.dot(x_ref[...], w_ref[...], preferred_element_type=F32)


def _prep_body(dp_ref, xw_ref, dis_ref, u_ref):
  deg = (jnp.max(dp_ref[0, :N, :], axis=1, keepdims=True)
         + jnp.max(dp_ref[1, :N, :], axis=1, keepdims=True) + 1.0)
  dis = lax.rsqrt(jnp.maximum(deg, 1e-12))
  dis_ref[...] = dis
  u_ref[...] = xw_ref[...] * dis


def _bn_relu(t, g, be):
  mu = jnp.mean(t, axis=0, keepdims=True)
  xc = t - mu
  var = jnp.mean(xc * xc, axis=0, keepdims=True)
  return jnp.maximum(xc * lax.rsqrt(var + 1e-5) * g + be, 0.0)


def _l1_body(p_ref, u1_ref, dis_ref, b_ref, g_ref, be_ref, u2_ref):
  dis = dis_ref[...]
  agg = (p_ref[0, :N, :].astype(F32) + p_ref[1, :N, :].astype(F32)
         + u1_ref[...])
  t = dis * agg + b_ref[...]
  u2_ref[...] = _bn_relu(t, g_ref[...], be_ref[...]) * dis


def _l2_body(q_ref, u2_ref, dis_ref, w2_ref, b2_ref, g2_ref, be2_ref,
             w3_ref, u3_ref):
  dis = dis_ref[...]
  aggpre = dis * (q_ref[0, :N, :].astype(F32) + q_ref[1, :N, :].astype(F32)
                  + u2_ref[...])
  t = jnp.dot(aggpre, w2_ref[...], preferred_element_type=F32) + b2_ref[...]
  h2 = _bn_relu(t, g2_ref[...], be2_ref[...])
  u3_ref[...] = jnp.dot(h2, w3_ref[...], preferred_element_type=F32) * dis


def _fin_body(r_ref, u3_ref, dis_ref, b3_ref, batch_ref, o_ref):
  h3 = dis_ref[...] * (r_ref[0, :N, :] + r_ref[1, :N, :] + u3_ref[...]) \
      + b3_ref[...]
  gid = lax.broadcasted_iota(jnp.int32, (N, G), 1)
  oh = (gid == batch_ref[...]).astype(F32)
  dn = (((0,), (0,)), ((), ()))
  sums = lax.dot_general(oh, h3, dn, preferred_element_type=F32)
  counts = lax.dot_general(oh, jnp.ones((N, 1), F32), dn,
                           preferred_element_type=F32)
  pooled = sums / jnp.maximum(counts, 1.0)
  lane = lax.broadcasted_iota(jnp.int32, (G, 16), 1)
  valid = lane < 10
  xm = jnp.where(valid, pooled, -1e30)
  mx = jnp.max(xm, axis=1, keepdims=True)
  ex = jnp.where(valid, jnp.exp(xm - mx), 0.0)
  lse = jnp.log(jnp.sum(ex, axis=1, keepdims=True))
  o_ref[...] = xm - mx - lse


def kernel(x, edge_index, batch, W1, b1, g1, be1, W2, b2, g2, be2, W3, b3):
  src = edge_index[0]
  dst = edge_index[1]
  pad = E_PAD - E
  srcp = jnp.concatenate([src, jnp.zeros((pad,), jnp.int32)]).reshape(-1, 256)
  dstp = jnp.concatenate([dst, jnp.full((pad,), TRASH, jnp.int32)]
                         ).reshape(-1, 256)

  dp = _deg_sc(dstp)
  xw1 = _tc(_mm_body, jax.ShapeDtypeStruct((N, 64), F32))(x, W1)
  dis, u1 = _tc(_prep_body, (jax.ShapeDtypeStruct((N, 1), F32),
                             jax.ShapeDtypeStruct((N, 64), F32)))(dp, xw1)
  p = _agg64_sc(u1.astype(jnp.bfloat16), srcp, dstp)
  u2 = _tc(_l1_body, jax.ShapeDtypeStruct((N, 64), F32))(
      p, u1, dis, b1.reshape(1, -1), g1.reshape(1, -1), be1.reshape(1, -1))
  q = _agg64_sc(u2.astype(jnp.bfloat16), srcp, dstp)
  w3p = jnp.pad(W3, ((0, 0), (0, 6)))
  u3 = _tc(_l2_body, jax.ShapeDtypeStruct((N, 16), F32))(
      q, u2, dis, W2, b2.reshape(1, -1), g2.reshape(1, -1),
      be2.reshape(1, -1), w3p)
  r = _agg16_sc(u3, srcp, dstp)
  b3p = jnp.pad(b3, (0, 6)).reshape(1, -1)
  out = _tc(_fin_body, jax.ShapeDtypeStruct((G, 16), F32))(
      r, u3, dis, b3p, batch[:, None])
  return out[:, :10]


# speedup vs baseline: 1.5550x; 1.0035x over previous
import functools

import jax
import jax.numpy as jnp
from jax import lax
from jax.experimental import pallas as pl
from jax.experimental.pallas import tpu as pltpu
from jax.experimental.pallas import tpu_sc as plsc

N = 10000
E = 320000
G = 64
NC, NS = 2, 16
NW = NC * NS
EPT = 10240
E_PAD = EPT * NW
R_TILE = 640
N_ROWS = NS * R_TILE
TRASH = N_ROWS - 8
ZROWS = 64
F32 = jnp.float32


def _sc_agg(D, gather, K, NBUF, dtype=F32):
  CPT = EPT // K
  lanes = 32 if dtype == jnp.bfloat16 else 16
  mesh = plsc.VectorSubcoreMesh(core_axis_name="c", subcore_axis_name="s")
  scratch = [
      pltpu.VMEM((CPT, K), jnp.int32),
      pltpu.VMEM((K, D), dtype),
      pltpu.VMEM((ZROWS, D), dtype),
      pltpu.VMEM_SHARED((N_ROWS, D), dtype),
      pltpu.SemaphoreType.DMA,
  ]
  if gather:
    scratch = ([pltpu.VMEM((CPT, K), jnp.int32)] + scratch
               + [pltpu.VMEM((K, D), dtype)] * (NBUF - 1)
               + [pltpu.SemaphoreType.DMA] * (2 * NBUF - 1))

  @functools.partial(
      pl.kernel,
      out_type=jax.ShapeDtypeStruct((NC, N_ROWS, D), dtype),
      mesh=mesh,
      scratch_types=scratch,
      compiler_params=pltpu.CompilerParams(use_tc_tiling_on_sc=False),
  )
  def body(*refs):
    if gather:
      (u_hbm, src_hbm, dst_hbm, out_hbm,
       idx_s, idx_d, buf, zbuf, acc, sem) = refs[:10]
      bufs = (buf,) + refs[10:10 + NBUF - 1]
      gsems = (sem,) + refs[10 + NBUF - 1:10 + 2 * (NBUF - 1)]
      ssems = refs[10 + 2 * (NBUF - 1):]
    else:
      dst_hbm, out_hbm, idx_d, buf, zbuf, acc, sem = refs
    c = lax.axis_index("c")
    s = lax.axis_index("s")
    wid = c * NS + s

    nvec = D // lanes

    def fill(ref, rows, val):
      def w(i, _):
        r = i // nvec
        col = (i % nvec) * lanes
        ref[r, pl.ds(col, lanes)] = jnp.full((lanes,), val, dtype)
        return 0
      lax.fori_loop(0, rows * nvec, w, 0)

    fill(zbuf, ZROWS, 0.0)
    def zcopy(k, _):
      pltpu.sync_copy(zbuf, acc.at[pl.ds(s * R_TILE + k * ZROWS, ZROWS)])
      return 0
    lax.fori_loop(0, R_TILE // ZROWS, zcopy, 0)
    if not gather:
      fill(buf, K, 1.0)
    pltpu.sync_copy(dst_hbm.at[pl.ds(wid * CPT, CPT)], idx_d)
    if gather:
      pltpu.sync_copy(src_hbm.at[pl.ds(wid * CPT, CPT)], idx_s)
    plsc.subcore_barrier()

    if gather:
      NG = CPT // NBUF

      def gath(b, t):
        return pltpu.make_async_copy(u_hbm.at[idx_s.at[b + t]],
                                     bufs[t], gsems[t])

      def scat(b, t):
        return pltpu.make_async_copy(bufs[t], acc.at[idx_d.at[b + t]],
                                     ssems[t])

      for t in range(NBUF):
        gath(0, t).start()

      def group(j, _):
        base = j * NBUF
        for t in range(NBUF):
          gath(base, t).wait()
          scat(base, t).start(add=True)

        @pl.when(j < NG - 1)
        def _():
          for t in range(NBUF):
            scat(base, t).wait()
            gath(base + NBUF, t).start()
        return 0
      lax.fori_loop(0, NG, group, 0)
      for t in range(NBUF):
        scat((NG - 1) * NBUF, t).wait()
    else:
      def chunk(i, _):
        pltpu.sync_copy(buf, acc.at[idx_d.at[i]], add=True)
        return 0
      lax.fori_loop(0, CPT, chunk, 0)
    plsc.subcore_barrier()
    pltpu.sync_copy(acc.at[pl.ds(s * R_TILE, R_TILE)],
                    out_hbm.at[c, pl.ds(s * R_TILE, R_TILE)])

  return body


_deg_sc = _sc_agg(16, gather=False, K=256, NBUF=1)
_agg64_sc = _sc_agg(64, gather=True, K=256, NBUF=4, dtype=jnp.bfloat16)
_agg16_sc = _sc_agg(16, gather=True, K=256, NBUF=8, dtype=jnp.bfloat16)


def _tc(body, out_shapes):
  return pl.pallas_call(body, out_shape=out_shapes)


def _mm_body(x_ref, w_ref, o_ref):
  o_ref[...] = jnp.dot(x_ref[...], w_ref[...], preferred_element_type=F32)


def _prep_body(dp_ref, xw_ref, dis_ref, u_ref):
  deg = (jnp.max(dp_ref[0, :N, :], axis=1, keepdims=True)
         + jnp.max(dp_ref[1, :N, :], axis=1, keepdims=True) + 1.0)
  dis = lax.rsqrt(jnp.maximum(deg, 1e-12))
  dis_ref[...] = dis
  u_ref[...] = xw_ref[...] * dis


def _bn_relu(t, g, be):
  mu = jnp.mean(t, axis=0, keepdims=True)
  xc = t - mu
  var = jnp.mean(xc * xc, axis=0, keepdims=True)
  return jnp.maximum(xc * lax.rsqrt(var + 1e-5) * g + be, 0.0)


def _l1_body(p_ref, u1_ref, dis_ref, b_ref, g_ref, be_ref, u2_ref):
  dis = dis_ref[...]
  agg = (p_ref[0, :N, :].astype(F32) + p_ref[1, :N, :].astype(F32)
         + u1_ref[...])
  t = dis * agg + b_ref[...]
  u2_ref[...] = _bn_relu(t, g_ref[...], be_ref[...]) * dis


def _l2_body(q_ref, u2_ref, dis_ref, w2_ref, b2_ref, g2_ref, be2_ref,
             w3_ref, u3_ref):
  dis = dis_ref[...]
  aggpre = dis * (q_ref[0, :N, :].astype(F32) + q_ref[1, :N, :].astype(F32)
                  + u2_ref[...])
  t = jnp.dot(aggpre, w2_ref[...], preferred_element_type=F32) + b2_ref[...]
  h2 = _bn_relu(t, g2_ref[...], be2_ref[...])
  u3_ref[...] = jnp.dot(h2, w3_ref[...], preferred_element_type=F32) * dis


def _fin_body(r_ref, u3_ref, dis_ref, b3_ref, batch_ref, o_ref):
  h3 = dis_ref[...] * (r_ref[0, :N, :].astype(F32)
                       + r_ref[1, :N, :].astype(F32) + u3_ref[...]) \
      + b3_ref[...]
  gid = lax.broadcasted_iota(jnp.int32, (N, G), 1)
  oh = (gid == batch_ref[...]).astype(F32)
  dn = (((0,), (0,)), ((), ()))
  sums = lax.dot_general(oh, h3, dn, preferred_element_type=F32)
  counts = lax.dot_general(oh, jnp.ones((N, 1), F32), dn,
                           preferred_element_type=F32)
  pooled = sums / jnp.maximum(counts, 1.0)
  lane = lax.broadcasted_iota(jnp.int32, (G, 16), 1)
  valid = lane < 10
  xm = jnp.where(valid, pooled, -1e30)
  mx = jnp.max(xm, axis=1, keepdims=True)
  ex = jnp.where(valid, jnp.exp(xm - mx), 0.0)
  lse = jnp.log(jnp.sum(ex, axis=1, keepdims=True))
  o_ref[...] = xm - mx - lse


def kernel(x, edge_index, batch, W1, b1, g1, be1, W2, b2, g2, be2, W3, b3):
  src = edge_index[0]
  dst = edge_index[1]
  pad = E_PAD - E
  srcp = jnp.concatenate([src, jnp.zeros((pad,), jnp.int32)]).reshape(-1, 256)
  dstp = jnp.concatenate([dst, jnp.full((pad,), TRASH, jnp.int32)]
                         ).reshape(-1, 256)

  dp = _deg_sc(dstp)
  xw1 = _tc(_mm_body, jax.ShapeDtypeStruct((N, 64), F32))(x, W1)
  dis, u1 = _tc(_prep_body, (jax.ShapeDtypeStruct((N, 1), F32),
                             jax.ShapeDtypeStruct((N, 64), F32)))(dp, xw1)
  p = _agg64_sc(u1.astype(jnp.bfloat16), srcp, dstp)
  u2 = _tc(_l1_body, jax.ShapeDtypeStruct((N, 64), F32))(
      p, u1, dis, b1.reshape(1, -1), g1.reshape(1, -1), be1.reshape(1, -1))
  q = _agg64_sc(u2.astype(jnp.bfloat16), srcp, dstp)
  w3p = jnp.pad(W3, ((0, 0), (0, 6)))
  u3 = _tc(_l2_body, jax.ShapeDtypeStruct((N, 16), F32))(
      q, u2, dis, W2, b2.reshape(1, -1), g2.reshape(1, -1),
      be2.reshape(1, -1), w3p)
  r = _agg16_sc(u3.astype(jnp.bfloat16), srcp, dstp)
  b3p = jnp.pad(b3, (0, 6)).reshape(1, -1)
  out = _tc(_fin_body, jax.ShapeDtypeStruct((G, 16), F32))(
      r, u3, dis, b3p, batch[:, None])
  return out[:, :10]
